# baseline (speedup 1.0000x reference)
import functools

import jax
import jax.numpy as jnp
from jax.experimental import pallas as pl
from jax.experimental.pallas import tpu as pltpu

B, C, V = 64, 4, 10000
DEG = 16
CL1_F, CL1_K = 32, 3
CL2_F, CL2_K = 64, 3
FC1_F, FC2_F = 512, 10
V2 = V // 4
FC1_IN = CL2_F * V // 16

FC_OCHUNK = 64
FC_STEPS = FC1_F // FC_OCHUNK


def _fc_body(h_ref, w1_ref, b1_ref, w2_ref, b2_ref, out_ref):
    k = pl.program_id(0)

    @pl.when(k == 0)
    def _init():
        out_ref[...] = jnp.broadcast_to(b2_ref[...], out_ref.shape)

    z = jax.nn.sigmoid(
        jax.lax.dot_general(h_ref[...], w1_ref[...], (((1,), (1,)), ((), ())),
                            preferred_element_type=jnp.float32)
        + b1_ref[0])
    out_ref[...] += jax.lax.dot_general(
        z, w2_ref[0], (((1,), (1,)), ((), ())),
        preferred_element_type=jnp.float32)


def _fc_head(h, FC1_w, FC1_b, FC2_w, FC2_b):
    return pl.pallas_call(
        _fc_body,
        grid=(FC_STEPS,),
        in_specs=[
            pl.BlockSpec((B, FC1_IN), lambda k: (0, 0)),
            pl.BlockSpec((FC_OCHUNK, FC1_IN), lambda k: (k, 0)),
            pl.BlockSpec((1, 1, FC_OCHUNK), lambda k: (k, 0, 0)),
            pl.BlockSpec((1, FC2_F, FC_OCHUNK), lambda k: (k, 0, 0)),
            pl.BlockSpec((1, FC2_F), lambda k: (0, 0)),
        ],
        out_specs=pl.BlockSpec((B, FC2_F), lambda k: (0, 0)),
        out_shape=jax.ShapeDtypeStruct((B, FC2_F), jnp.float32),
    )(h, FC1_w,
      FC1_b.reshape(FC_STEPS, 1, FC_OCHUNK),
      FC2_w.reshape(FC2_F, FC_STEPS, FC_OCHUNK).transpose(1, 0, 2),
      FC2_b.reshape(1, FC2_F))


def _spmm(rows, cols, vals, X, n):
    return jax.ops.segment_sum(vals[:, None] * jnp.take(X, cols, axis=0), rows,
                               num_segments=n)


def _graph_conv(x, rows, cols, vals, W, b, K):
    Bb, Cc, Vv = x.shape
    x0 = jnp.transpose(x, (2, 1, 0)).reshape(Vv, Cc * Bb)
    xs = [x0]
    if K > 1:
        xs.append(_spmm(rows, cols, vals, x0, Vv))
    for k in range(2, K):
        xs.append(2.0 * _spmm(rows, cols, vals, xs[-1], Vv) - xs[-2])
    X = jnp.stack(xs, axis=0).reshape(K, Vv, Cc, Bb)
    X = jnp.transpose(X, (3, 1, 2, 0)).reshape(Bb * Vv, Cc * K)
    out = X @ W.T + b
    out = out.reshape(Bb, Vv, -1)
    return jnp.transpose(out, (0, 2, 1))


def _maxpool4(x):
    Bb, Ff, Vv = x.shape
    return x.reshape(Bb, Ff, Vv // 4, 4).max(axis=-1)


def kernel(x, rows1, cols1, vals1, rows2, cols2, vals2,
           GCL1_w, GCL1_b, GCL2_w, GCL2_b, FC1_w, FC1_b, FC2_w, FC2_b):
    h = x / jnp.sqrt(1.0 + 1e-5)
    h = _graph_conv(h, rows1, cols1, vals1, GCL1_w, GCL1_b, CL1_K)
    h = jax.nn.relu(h)
    h = _maxpool4(h)
    h = _graph_conv(h, rows2, cols2, vals2, GCL2_w, GCL2_b, CL2_K)
    h = jax.nn.relu(h)
    h = _maxpool4(h)
    h = h.reshape(h.shape[0], -1)
    return _fc_head(h, FC1_w, FC1_b, FC2_w, FC2_b)
